# baseline (speedup 1.0000x reference)
import functools

import jax
import jax.numpy as jnp
from jax import lax
from jax.experimental import pallas as pl
from jax.experimental.pallas import tpu as pltpu
from jax.experimental.pallas import tpu_sc as plsc

F32 = jnp.float32
I32 = jnp.int32

NC = 2
NS = 16
NW = NC * NS
CH = 128

_MESH = plsc.VectorSubcoreMesh(
    core_axis_name="c", subcore_axis_name="s", num_cores=NC, num_subcores=NS)


def _mk_deg_kernel(npad, kch):
    rpt = npad // NS

    @functools.partial(
        pl.kernel,
        out_type=jax.ShapeDtypeStruct((NC * npad,), F32),
        mesh=_MESH,
        scratch_types=[
            pltpu.VMEM((kch, CH), I32),
            pltpu.VMEM((CH,), F32),
            pltpu.VMEM((rpt,), F32),
            pltpu.VMEM_SHARED((npad,), F32),
        ],
    )
    def deg_kernel(dstp_hbm, deg_out, dst_v, ones_v, zbuf, acc_sh):
        cid = lax.axis_index("c")
        sid = lax.axis_index("s")
        wid = cid * NS + sid
        pltpu.sync_copy(dstp_hbm.at[wid], dst_v)
        for i in range(CH // 16):
            ones_v[pl.ds(i * 16, 16)] = jnp.full((16,), 1.0, F32)

        @pl.loop(0, rpt // 16)
        def _(i):
            zbuf[pl.ds(i * 16, 16)] = jnp.zeros((16,), F32)

        pltpu.sync_copy(zbuf, acc_sh.at[pl.ds(sid * rpt, rpt)])
        plsc.subcore_barrier()

        @pl.loop(0, kch)
        def _(j):
            pltpu.sync_copy(ones_v, acc_sh.at[dst_v.at[j]], add=True)

        plsc.subcore_barrier()
        pltpu.sync_copy(acc_sh.at[pl.ds(sid * rpt, rpt)],
                        deg_out.at[pl.ds(cid * npad + sid * rpt, rpt)])

    return deg_kernel


def _mk_scatter_kernel(npad, kch):
    rpt = npad // NS

    @functools.partial(
        pl.kernel,
        out_type=jax.ShapeDtypeStruct((NC, npad, CH), F32),
        mesh=_MESH,
        scratch_types=[
            pltpu.VMEM((kch, CH), I32),
            pltpu.VMEM((kch, CH), I32),
            pltpu.VMEM((CH, CH), F32),
            pltpu.VMEM_SHARED((npad, CH), F32),
        ],
    )
    def scatter_kernel(hs_hbm, srcp_hbm, dstp_hbm, acc_out,
                       src_v, dst_v, buf0, acc_sh):
        cid = lax.axis_index("c")
        sid = lax.axis_index("s")
        wid = cid * NS + sid
        pltpu.sync_copy(srcp_hbm.at[wid], src_v)
        pltpu.sync_copy(dstp_hbm.at[wid], dst_v)

        @pl.loop(0, CH)
        def _(i):
            for k in range(CH // 16):
                buf0[i, pl.ds(k * 16, 16)] = jnp.zeros((16,), F32)

        @pl.loop(0, rpt // CH)
        def _(t):
            pltpu.sync_copy(buf0, acc_sh.at[pl.ds(sid * rpt + t * CH, CH)])

        plsc.subcore_barrier()

        @pl.loop(0, kch)
        def _(j):
            pltpu.sync_copy(hs_hbm.at[src_v.at[j]], buf0)
            pltpu.sync_copy(buf0, acc_sh.at[dst_v.at[j]], add=True)

        plsc.subcore_barrier()
        pltpu.sync_copy(acc_sh.at[pl.ds(sid * rpt, rpt)],
                        acc_out.at[cid, pl.ds(sid * rpt, rpt)])

    return scatter_kernel


def _tc1_body(x_ref, w_ref, degt_ref, hs_ref, dis_ref):
    deg = degt_ref[:, 0:1] + degt_ref[:, 1:2] + 1.0
    dis = lax.rsqrt(deg)
    dis_ref[...] = dis
    hs_ref[...] = jnp.dot(x_ref[...], w_ref[...],
                          preferred_element_type=F32) * dis


def _tc2_body(acc_ref, hs_ref, dis_ref, b_ref, w_ref, out_ref):
    dis = dis_ref[...]
    p = acc_ref[0] + acc_ref[1] + hs_ref[...]
    h = jnp.maximum(p * dis + b_ref[...], 0.0)
    out_ref[...] = jnp.dot(h, w_ref[...], preferred_element_type=F32) * dis


def _tc3_body(acc_ref, hs_ref, dis_ref, b_ref, w_ref, bfc_ref, out_ref):
    p = acc_ref[0] + acc_ref[1] + hs_ref[...]
    h = jnp.maximum(p * dis_ref[...] + b_ref[...], 0.0)
    out_ref[...] = jnp.dot(h, w_ref[...], preferred_element_type=F32) + bfc_ref[...]


def kernel(x, edge_index, W1, b1, W2, b2, Wfc, bfc):
    n, c = x.shape
    e = edge_index.shape[1]

    kch = -(-e // (NW * CH))
    epw = kch * CH
    etot = NW * epw
    npad = -(-(n + 1) // (NS * 16)) * (NS * 16)
    rpt = npad // NS
    rblk = rpt // 2

    ei = edge_index.astype(I32)
    nspare = npad - n
    pad = n + jnp.arange(etot - e, dtype=I32) % nspare
    srcp = jnp.concatenate([ei[0], pad]).reshape(NW, kch, CH)
    dstp = jnp.concatenate([ei[1], pad]).reshape(NW, kch, CH)
    xp = jnp.zeros((npad, c), F32).at[:n].set(x)

    deg_kernel = _mk_deg_kernel(npad, kch)
    scatter_kernel = _mk_scatter_kernel(npad, kch)

    degp = deg_kernel(dstp)
    degt = degp.reshape(NC, npad).T

    grid = (npad // rblk,)
    row_spec = pl.BlockSpec((rblk, CH), lambda i: (i, 0))
    mat_spec = pl.BlockSpec((CH, CH), lambda i: (0, 0))
    vec_spec = pl.BlockSpec((1, CH), lambda i: (0, 0))
    col_spec = pl.BlockSpec((rblk, 1), lambda i: (i, 0))
    acc_spec = pl.BlockSpec((NC, rblk, CH), lambda i: (0, i, 0))
    degt_spec = pl.BlockSpec((rblk, NC), lambda i: (i, 0))

    hs1, dis = pl.pallas_call(
        _tc1_body,
        grid=grid,
        in_specs=[row_spec, mat_spec, degt_spec],
        out_specs=[row_spec, col_spec],
        out_shape=[jax.ShapeDtypeStruct((npad, c), F32),
                   jax.ShapeDtypeStruct((npad, 1), F32)],
    )(xp, W1, degt)

    acc1 = scatter_kernel(hs1, srcp, dstp)

    hs2 = pl.pallas_call(
        _tc2_body,
        grid=grid,
        in_specs=[acc_spec, row_spec, col_spec, vec_spec, mat_spec],
        out_specs=row_spec,
        out_shape=jax.ShapeDtypeStruct((npad, c), F32),
    )(acc1, hs1, dis, b1.reshape(1, c), W2)

    acc2 = scatter_kernel(hs2, srcp, dstp)

    out = pl.pallas_call(
        _tc3_body,
        grid=grid,
        in_specs=[acc_spec, row_spec, col_spec, vec_spec, mat_spec, vec_spec],
        out_specs=row_spec,
        out_shape=jax.ShapeDtypeStruct((n, c), F32),
    )(acc2, hs2, dis, b2.reshape(1, c), Wfc, bfc.reshape(1, c))

    return out


# speedup vs baseline: 1.0552x; 1.0552x over previous
import functools

import jax
import jax.numpy as jnp
from jax import lax
from jax.experimental import pallas as pl
from jax.experimental.pallas import tpu as pltpu
from jax.experimental.pallas import tpu_sc as plsc

F32 = jnp.float32
I32 = jnp.int32

NC = 2
NS = 16
NW = NC * NS
CH = 128

_MESH = plsc.VectorSubcoreMesh(
    core_axis_name="c", subcore_axis_name="s", num_cores=NC, num_subcores=NS)


def _mk_deg_kernel(npad, kch):
    rpt = npad // NS

    @functools.partial(
        pl.kernel,
        out_type=jax.ShapeDtypeStruct((NC * npad,), F32),
        mesh=_MESH,
        scratch_types=[
            pltpu.VMEM((kch, CH), I32),
            pltpu.VMEM((CH,), F32),
            pltpu.VMEM((rpt,), F32),
            pltpu.VMEM_SHARED((npad,), F32),
        ],
    )
    def deg_kernel(dstp_hbm, deg_out, dst_v, ones_v, zbuf, acc_sh):
        cid = lax.axis_index("c")
        sid = lax.axis_index("s")
        wid = cid * NS + sid
        pltpu.sync_copy(dstp_hbm.at[wid], dst_v)
        for i in range(CH // 16):
            ones_v[pl.ds(i * 16, 16)] = jnp.full((16,), 1.0, F32)

        @pl.loop(0, rpt // 16)
        def _(i):
            zbuf[pl.ds(i * 16, 16)] = jnp.zeros((16,), F32)

        pltpu.sync_copy(zbuf, acc_sh.at[pl.ds(sid * rpt, rpt)])
        plsc.subcore_barrier()

        @pl.loop(0, kch)
        def _(j):
            pltpu.sync_copy(ones_v, acc_sh.at[dst_v.at[j]], add=True)

        plsc.subcore_barrier()
        pltpu.sync_copy(acc_sh.at[pl.ds(sid * rpt, rpt)],
                        deg_out.at[pl.ds(cid * npad + sid * rpt, rpt)])

    return deg_kernel


def _mk_scatter_kernel(npad, kch):
    rpt = npad // NS

    @functools.partial(
        pl.kernel,
        out_type=jax.ShapeDtypeStruct((NC, npad, CH), F32),
        mesh=_MESH,
        scratch_types=[
            pltpu.VMEM((kch, CH), I32),
            pltpu.VMEM((kch, CH), I32),
            pltpu.VMEM((CH, CH), F32),
            pltpu.VMEM_SHARED((npad, CH), F32),
        ],
    )
    def scatter_kernel(hs_hbm, srcp_hbm, dstp_hbm, acc_out,
                       src_v, dst_v, buf0, acc_sh):
        cid = lax.axis_index("c")
        sid = lax.axis_index("s")
        wid = cid * NS + sid
        pltpu.sync_copy(srcp_hbm.at[wid], src_v)
        pltpu.sync_copy(dstp_hbm.at[wid], dst_v)

        @pl.loop(0, CH)
        def _(i):
            for k in range(CH // 16):
                buf0[i, pl.ds(k * 16, 16)] = jnp.zeros((16,), F32)

        @pl.loop(0, rpt // CH)
        def _(t):
            pltpu.sync_copy(buf0, acc_sh.at[pl.ds(sid * rpt + t * CH, CH)])

        plsc.subcore_barrier()

        @pl.loop(0, kch)
        def _(j):
            pltpu.sync_copy(hs_hbm.at[src_v.at[j]], buf0)
            pltpu.sync_copy(buf0, acc_sh.at[dst_v.at[j]], add=True)

        plsc.subcore_barrier()
        pltpu.sync_copy(acc_sh.at[pl.ds(sid * rpt, rpt)],
                        acc_out.at[cid, pl.ds(sid * rpt, rpt)])

    return scatter_kernel


def _tc1_body(x_ref, w_ref, degt_ref, hs_ref, dis_ref):
    deg = degt_ref[:, 0:1] + degt_ref[:, 1:2] + 1.0
    dis = lax.rsqrt(deg)
    dis_ref[...] = dis
    hs_ref[...] = jnp.dot(x_ref[...], w_ref[...],
                          preferred_element_type=F32) * dis


def _tc2_body(acc_ref, hs_ref, dis_ref, b_ref, w_ref, out_ref):
    dis = dis_ref[...]
    p = acc_ref[0] + acc_ref[1] + hs_ref[...]
    h = jnp.maximum(p * dis + b_ref[...], 0.0)
    out_ref[...] = jnp.dot(h, w_ref[...], preferred_element_type=F32) * dis


def _tc3_body(acc_ref, hs_ref, dis_ref, b_ref, w_ref, bfc_ref, out_ref):
    p = acc_ref[0] + acc_ref[1] + hs_ref[...]
    h = jnp.maximum(p * dis_ref[...] + b_ref[...], 0.0)
    out_ref[...] = jnp.dot(h, w_ref[...], preferred_element_type=F32) + bfc_ref[...]


def kernel(x, edge_index, W1, b1, W2, b2, Wfc, bfc):
    n, c = x.shape
    e = edge_index.shape[1]

    kch = -(-e // (NW * CH))
    epw = kch * CH
    etot = NW * epw
    npad = -(-(n + 1) // (NS * 16)) * (NS * 16)
    rpt = npad // NS
    rblk = rpt

    ei = edge_index.astype(I32)
    nspare = npad - n
    pad = n + jnp.arange(etot - e, dtype=I32) % nspare
    srcp = jnp.concatenate([ei[0], pad]).reshape(NW, kch, CH)
    dstp = jnp.concatenate([ei[1], pad]).reshape(NW, kch, CH)
    xp = jnp.zeros((npad, c), F32).at[:n].set(x)

    deg_kernel = _mk_deg_kernel(npad, kch)
    scatter_kernel = _mk_scatter_kernel(npad, kch)

    degp = deg_kernel(dstp)
    degt = degp.reshape(NC, npad).T

    grid = (npad // rblk,)
    row_spec = pl.BlockSpec((rblk, CH), lambda i: (i, 0))
    mat_spec = pl.BlockSpec((CH, CH), lambda i: (0, 0))
    vec_spec = pl.BlockSpec((1, CH), lambda i: (0, 0))
    col_spec = pl.BlockSpec((rblk, 1), lambda i: (i, 0))
    acc_spec = pl.BlockSpec((NC, rblk, CH), lambda i: (0, i, 0))
    degt_spec = pl.BlockSpec((rblk, NC), lambda i: (i, 0))

    hs1, dis = pl.pallas_call(
        _tc1_body,
        grid=grid,
        in_specs=[row_spec, mat_spec, degt_spec],
        out_specs=[row_spec, col_spec],
        out_shape=[jax.ShapeDtypeStruct((npad, c), F32),
                   jax.ShapeDtypeStruct((npad, 1), F32)],
    )(xp, W1, degt)

    acc1 = scatter_kernel(hs1, srcp, dstp)

    hs2 = pl.pallas_call(
        _tc2_body,
        grid=grid,
        in_specs=[acc_spec, row_spec, col_spec, vec_spec, mat_spec],
        out_specs=row_spec,
        out_shape=jax.ShapeDtypeStruct((npad, c), F32),
    )(acc1, hs1, dis, b1.reshape(1, c), W2)

    acc2 = scatter_kernel(hs2, srcp, dstp)

    out = pl.pallas_call(
        _tc3_body,
        grid=grid,
        in_specs=[acc_spec, row_spec, col_spec, vec_spec, mat_spec, vec_spec],
        out_specs=row_spec,
        out_shape=jax.ShapeDtypeStruct((n, c), F32),
    )(acc2, hs2, dis, b2.reshape(1, c), Wfc, bfc.reshape(1, c))

    return out


# speedup vs baseline: 1.0895x; 1.0326x over previous
import functools

import jax
import jax.numpy as jnp
from jax import lax
from jax.experimental import pallas as pl
from jax.experimental.pallas import tpu as pltpu
from jax.experimental.pallas import tpu_sc as plsc

F32 = jnp.float32
I32 = jnp.int32

NC = 2
NS = 16
NW = NC * NS
CH = 128

_MESH = plsc.VectorSubcoreMesh(
    core_axis_name="c", subcore_axis_name="s", num_cores=NC, num_subcores=NS)


def _mk_deg_kernel(npad, kch):
    rpt = npad // NS

    @functools.partial(
        pl.kernel,
        out_type=jax.ShapeDtypeStruct((NC * npad,), F32),
        mesh=_MESH,
        scratch_types=[
            pltpu.VMEM((kch, CH), I32),
            pltpu.VMEM((CH,), F32),
            pltpu.VMEM((rpt,), F32),
            pltpu.VMEM_SHARED((npad,), F32),
        ],
    )
    def deg_kernel(dstp_hbm, deg_out, dst_v, ones_v, zbuf, acc_sh):
        cid = lax.axis_index("c")
        sid = lax.axis_index("s")
        wid = cid * NS + sid
        pltpu.sync_copy(dstp_hbm.at[wid], dst_v)
        for i in range(CH // 16):
            ones_v[pl.ds(i * 16, 16)] = jnp.full((16,), 1.0, F32)

        @pl.loop(0, rpt // 16)
        def _(i):
            zbuf[pl.ds(i * 16, 16)] = jnp.zeros((16,), F32)

        pltpu.sync_copy(zbuf, acc_sh.at[pl.ds(sid * rpt, rpt)])
        plsc.subcore_barrier()

        @pl.loop(0, kch)
        def _(j):
            pltpu.sync_copy(ones_v, acc_sh.at[dst_v.at[j]], add=True)

        plsc.subcore_barrier()
        pltpu.sync_copy(acc_sh.at[pl.ds(sid * rpt, rpt)],
                        deg_out.at[pl.ds(cid * npad + sid * rpt, rpt)])

    return deg_kernel


def _mk_scatter_kernel(npad, kch):
    rpt = npad // NS

    @functools.partial(
        pl.kernel,
        out_type=jax.ShapeDtypeStruct((NC, npad, CH), F32),
        mesh=_MESH,
        scratch_types=[
            pltpu.VMEM((kch, CH), I32),
            pltpu.VMEM((kch, CH), I32),
            pltpu.VMEM((CH, CH), F32),
            pltpu.VMEM_SHARED((npad, CH), F32),
        ],
    )
    def scatter_kernel(hs_hbm, srcp_hbm, dstp_hbm, acc_out,
                       src_v, dst_v, buf0, acc_sh):
        cid = lax.axis_index("c")
        sid = lax.axis_index("s")
        wid = cid * NS + sid
        pltpu.sync_copy(srcp_hbm.at[wid], src_v)
        pltpu.sync_copy(dstp_hbm.at[wid], dst_v)

        @pl.loop(0, CH)
        def _(i):
            for k in range(CH // 16):
                buf0[i, pl.ds(k * 16, 16)] = jnp.zeros((16,), F32)

        @pl.loop(0, rpt // CH)
        def _(t):
            pltpu.sync_copy(buf0, acc_sh.at[pl.ds(sid * rpt + t * CH, CH)])

        plsc.subcore_barrier()

        @pl.loop(0, kch)
        def _(j):
            pltpu.sync_copy(hs_hbm.at[src_v.at[j]], buf0)
            pltpu.sync_copy(buf0, acc_sh.at[dst_v.at[j]], add=True)

        plsc.subcore_barrier()
        pltpu.sync_copy(acc_sh.at[pl.ds(sid * rpt, rpt)],
                        acc_out.at[cid, pl.ds(sid * rpt, rpt)])

    return scatter_kernel


def _tc1_body(x_ref, w_ref, degt_ref, hs_ref, dis_ref):
    deg = degt_ref[:, 0:1] + degt_ref[:, 1:2] + 1.0
    dis = lax.rsqrt(deg)
    dis_ref[...] = dis
    hs_ref[...] = jnp.dot(x_ref[...], w_ref[...],
                          preferred_element_type=F32) * dis


def _tc2_body(acc_ref, hs_ref, dis_ref, b_ref, w_ref, out_ref):
    dis = dis_ref[...]
    p = acc_ref[0] + acc_ref[1] + hs_ref[...]
    h = jnp.maximum(p * dis + b_ref[...], 0.0)
    out_ref[...] = jnp.dot(h, w_ref[...], preferred_element_type=F32) * dis


def _tc3_body(acc_ref, hs_ref, dis_ref, b_ref, w_ref, bfc_ref, out_ref):
    p = acc_ref[0] + acc_ref[1] + hs_ref[...]
    h = jnp.maximum(p * dis_ref[...] + b_ref[...], 0.0)
    out_ref[...] = jnp.dot(h, w_ref[...], preferred_element_type=F32) + bfc_ref[...]


def kernel(x, edge_index, W1, b1, W2, b2, Wfc, bfc):
    n, c = x.shape
    e = edge_index.shape[1]

    kch = -(-e // (NW * CH))
    epw = kch * CH
    etot = NW * epw
    npad = -(-(n + 1) // (NS * 16)) * (NS * 16)
    rpt = npad // NS
    rblk = rpt * 2

    ei = edge_index.astype(I32)
    nspare = npad - n
    pad = n + jnp.arange(etot - e, dtype=I32) % nspare
    srcp = jnp.concatenate([ei[0], pad]).reshape(NW, kch, CH)
    dstp = jnp.concatenate([ei[1], pad]).reshape(NW, kch, CH)
    xp = jnp.zeros((npad, c), F32).at[:n].set(x)

    deg_kernel = _mk_deg_kernel(npad, kch)
    scatter_kernel = _mk_scatter_kernel(npad, kch)

    degp = deg_kernel(dstp)
    degt = degp.reshape(NC, npad).T

    grid = (npad // rblk,)
    row_spec = pl.BlockSpec((rblk, CH), lambda i: (i, 0))
    mat_spec = pl.BlockSpec((CH, CH), lambda i: (0, 0))
    vec_spec = pl.BlockSpec((1, CH), lambda i: (0, 0))
    col_spec = pl.BlockSpec((rblk, 1), lambda i: (i, 0))
    acc_spec = pl.BlockSpec((NC, rblk, CH), lambda i: (0, i, 0))
    degt_spec = pl.BlockSpec((rblk, NC), lambda i: (i, 0))

    hs1, dis = pl.pallas_call(
        _tc1_body,
        grid=grid,
        in_specs=[row_spec, mat_spec, degt_spec],
        out_specs=[row_spec, col_spec],
        out_shape=[jax.ShapeDtypeStruct((npad, c), F32),
                   jax.ShapeDtypeStruct((npad, 1), F32)],
    )(xp, W1, degt)

    acc1 = scatter_kernel(hs1, srcp, dstp)

    hs2 = pl.pallas_call(
        _tc2_body,
        grid=grid,
        in_specs=[acc_spec, row_spec, col_spec, vec_spec, mat_spec],
        out_specs=row_spec,
        out_shape=jax.ShapeDtypeStruct((npad, c), F32),
    )(acc1, hs1, dis, b1.reshape(1, c), W2)

    acc2 = scatter_kernel(hs2, srcp, dstp)

    out = pl.pallas_call(
        _tc3_body,
        grid=grid,
        in_specs=[acc_spec, row_spec, col_spec, vec_spec, mat_spec, vec_spec],
        out_specs=row_spec,
        out_shape=jax.ShapeDtypeStruct((n, c), F32),
    )(acc2, hs2, dis, b2.reshape(1, c), Wfc, bfc.reshape(1, c))

    return out


# speedup vs baseline: 1.1025x; 1.0119x over previous
import functools

import jax
import jax.numpy as jnp
from jax import lax
from jax.experimental import pallas as pl
from jax.experimental.pallas import tpu as pltpu
from jax.experimental.pallas import tpu_sc as plsc

F32 = jnp.float32
I32 = jnp.int32

NC = 2
NS = 16
NW = NC * NS
CH = 128

_MESH = plsc.VectorSubcoreMesh(
    core_axis_name="c", subcore_axis_name="s", num_cores=NC, num_subcores=NS)


def _mk_deg_kernel(npad, kch):
    rpt = npad // NS

    @functools.partial(
        pl.kernel,
        out_type=jax.ShapeDtypeStruct((NC * npad,), F32),
        mesh=_MESH,
        scratch_types=[
            pltpu.VMEM((kch, CH), I32),
            pltpu.VMEM((CH,), F32),
            pltpu.VMEM((rpt,), F32),
            pltpu.VMEM_SHARED((npad,), F32),
        ],
    )
    def deg_kernel(dstp_hbm, deg_out, dst_v, ones_v, zbuf, acc_sh):
        cid = lax.axis_index("c")
        sid = lax.axis_index("s")
        wid = cid * NS + sid
        pltpu.sync_copy(dstp_hbm.at[wid], dst_v)
        for i in range(CH // 16):
            ones_v[pl.ds(i * 16, 16)] = jnp.full((16,), 1.0, F32)

        @pl.loop(0, rpt // 16)
        def _(i):
            zbuf[pl.ds(i * 16, 16)] = jnp.zeros((16,), F32)

        pltpu.sync_copy(zbuf, acc_sh.at[pl.ds(sid * rpt, rpt)])
        plsc.subcore_barrier()

        @pl.loop(0, kch)
        def _(j):
            pltpu.sync_copy(ones_v, acc_sh.at[dst_v.at[j]], add=True)

        plsc.subcore_barrier()
        pltpu.sync_copy(acc_sh.at[pl.ds(sid * rpt, rpt)],
                        deg_out.at[pl.ds(cid * npad + sid * rpt, rpt)])

    return deg_kernel


def _mk_scatter_kernel(npad, kch):
    rpt = npad // NS

    @functools.partial(
        pl.kernel,
        out_type=jax.ShapeDtypeStruct((NC, npad, CH), F32),
        mesh=_MESH,
        scratch_types=[
            pltpu.VMEM((kch, CH), I32),
            pltpu.VMEM((kch, CH), I32),
            pltpu.VMEM((CH, CH), F32),
            pltpu.VMEM_SHARED((npad, CH), F32),
        ],
    )
    def scatter_kernel(hs_hbm, srcp_hbm, dstp_hbm, acc_out,
                       src_v, dst_v, buf0, acc_sh):
        cid = lax.axis_index("c")
        sid = lax.axis_index("s")
        wid = cid * NS + sid
        pltpu.sync_copy(srcp_hbm.at[wid], src_v)
        pltpu.sync_copy(dstp_hbm.at[wid], dst_v)

        @pl.loop(0, CH)
        def _(i):
            for k in range(CH // 16):
                buf0[i, pl.ds(k * 16, 16)] = jnp.zeros((16,), F32)

        @pl.loop(0, rpt // CH)
        def _(t):
            pltpu.sync_copy(buf0, acc_sh.at[pl.ds(sid * rpt + t * CH, CH)])

        plsc.subcore_barrier()

        @pl.loop(0, kch)
        def _(j):
            pltpu.sync_copy(hs_hbm.at[src_v.at[j]], buf0)
            pltpu.sync_copy(buf0, acc_sh.at[dst_v.at[j]], add=True)

        plsc.subcore_barrier()
        pltpu.sync_copy(acc_sh.at[pl.ds(sid * rpt, rpt)],
                        acc_out.at[cid, pl.ds(sid * rpt, rpt)])

    return scatter_kernel


def _tc1_body(x_ref, w_ref, degt_ref, hs_ref, dis_ref):
    deg = degt_ref[:, 0:1] + degt_ref[:, 1:2] + 1.0
    dis = lax.rsqrt(deg)
    dis_ref[...] = dis
    hs_ref[...] = jnp.dot(x_ref[...], w_ref[...],
                          preferred_element_type=F32) * dis


def _tc2_body(acc_ref, hs_ref, dis_ref, b_ref, w_ref, out_ref):
    dis = dis_ref[...]
    p = acc_ref[0] + acc_ref[1] + hs_ref[...]
    h = jnp.maximum(p * dis + b_ref[...], 0.0)
    out_ref[...] = jnp.dot(h, w_ref[...], preferred_element_type=F32) * dis


def _tc3_body(acc_ref, hs_ref, dis_ref, b_ref, w_ref, bfc_ref, out_ref):
    p = acc_ref[0] + acc_ref[1] + hs_ref[...]
    h = jnp.maximum(p * dis_ref[...] + b_ref[...], 0.0)
    out_ref[...] = jnp.dot(h, w_ref[...], preferred_element_type=F32) + bfc_ref[...]


def kernel(x, edge_index, W1, b1, W2, b2, Wfc, bfc):
    n, c = x.shape
    e = edge_index.shape[1]

    kch = -(-e // (NW * CH))
    epw = kch * CH
    etot = NW * epw
    npad = -(-(n + 1) // (NS * 16)) * (NS * 16)
    rpt = npad // NS
    rblk = rpt * 4

    ei = edge_index.astype(I32)
    nspare = npad - n
    pad = n + jnp.arange(etot - e, dtype=I32) % nspare
    srcp = jnp.concatenate([ei[0], pad]).reshape(NW, kch, CH)
    dstp = jnp.concatenate([ei[1], pad]).reshape(NW, kch, CH)
    xp = jnp.zeros((npad, c), F32).at[:n].set(x)

    deg_kernel = _mk_deg_kernel(npad, kch)
    scatter_kernel = _mk_scatter_kernel(npad, kch)

    degp = deg_kernel(dstp)
    degt = degp.reshape(NC, npad).T

    grid = (npad // rblk,)
    row_spec = pl.BlockSpec((rblk, CH), lambda i: (i, 0))
    mat_spec = pl.BlockSpec((CH, CH), lambda i: (0, 0))
    vec_spec = pl.BlockSpec((1, CH), lambda i: (0, 0))
    col_spec = pl.BlockSpec((rblk, 1), lambda i: (i, 0))
    acc_spec = pl.BlockSpec((NC, rblk, CH), lambda i: (0, i, 0))
    degt_spec = pl.BlockSpec((rblk, NC), lambda i: (i, 0))

    hs1, dis = pl.pallas_call(
        _tc1_body,
        grid=grid,
        in_specs=[row_spec, mat_spec, degt_spec],
        out_specs=[row_spec, col_spec],
        out_shape=[jax.ShapeDtypeStruct((npad, c), F32),
                   jax.ShapeDtypeStruct((npad, 1), F32)],
    )(xp, W1, degt)

    acc1 = scatter_kernel(hs1, srcp, dstp)

    hs2 = pl.pallas_call(
        _tc2_body,
        grid=grid,
        in_specs=[acc_spec, row_spec, col_spec, vec_spec, mat_spec],
        out_specs=row_spec,
        out_shape=jax.ShapeDtypeStruct((npad, c), F32),
    )(acc1, hs1, dis, b1.reshape(1, c), W2)

    acc2 = scatter_kernel(hs2, srcp, dstp)

    out = pl.pallas_call(
        _tc3_body,
        grid=grid,
        in_specs=[acc_spec, row_spec, col_spec, vec_spec, mat_spec, vec_spec],
        out_specs=row_spec,
        out_shape=jax.ShapeDtypeStruct((n, c), F32),
    )(acc2, hs2, dis, b2.reshape(1, c), Wfc, bfc.reshape(1, c))

    return out
